# baseline (device time: 19803 ns/iter reference)
import jax
import jax.numpy as jnp
from jax import lax
from jax.experimental import pallas as pl
from jax.experimental.pallas import tpu as pltpu

N_DEV = 4
B = 2
SQ = 256
SKV = 256
HQ = 4
DH = 64
D = 512
HALO = 128
WIN = 128
KW = SKV + 2 * HALO


def _body(x_ref, wq_ref, k_ref, v_ref, kf_ref, kl_ref, vf_ref, vl_ref,
          wo_ref, out_ref, kleft, kright, vleft, vright,
          send_sems, recv_sems):
    pos = lax.axis_index("i")
    left = (pos - 1) % N_DEV
    right = (pos + 1) % N_DEV

    barrier_sem = pltpu.get_barrier_semaphore()
    for nbr in (left, right):
        pl.semaphore_signal(
            barrier_sem, inc=1,
            device_id=(nbr,), device_id_type=pl.DeviceIdType.MESH,
        )
    pl.semaphore_wait(barrier_sem, 2)

    transfers = (
        (kl_ref, kleft, right),
        (vl_ref, vleft, right),
        (kf_ref, kright, left),
        (vf_ref, vright, left),
    )
    rdmas = []
    for idx, (src, dst, tgt) in enumerate(transfers):
        rdma = pltpu.make_async_remote_copy(
            src_ref=src, dst_ref=dst,
            send_sem=send_sems.at[idx], recv_sem=recv_sems.at[idx],
            device_id=(tgt,), device_id_type=pl.DeviceIdType.MESH,
        )
        rdma.start()
        rdmas.append(rdma)

    q = []
    for b in range(B):
        qb = lax.dot_general(
            x_ref[b], wq_ref[...],
            (((1,), (0,)), ((), ())), preferred_element_type=jnp.float32,
        )
        q.append(qb.astype(jnp.bfloat16))

    for rdma in rdmas:
        rdma.wait()

    i2 = lax.broadcasted_iota(jnp.int32, (SQ, KW), 0)
    j2 = lax.broadcasted_iota(jnp.int32, (SQ, KW), 1)
    ki = pos * SKV - HALO + j2
    mask = (jnp.abs(i2 + HALO - j2) <= WIN) & (ki >= 0) & (ki < N_DEV * SKV)

    for b in range(B):
        ctx_heads = []
        for h in range(HQ):
            bh = b * HQ + h
            qh = q[b][:, h * DH:(h + 1) * DH]
            kfull = jnp.concatenate(
                [kleft[bh], k_ref[bh], kright[bh]], axis=0)
            vfull = jnp.concatenate(
                [vleft[bh], v_ref[bh], vright[bh]], axis=0)
            s = lax.dot_general(
                qh, kfull, (((1,), (1,)), ((), ())),
                preferred_element_type=jnp.float32,
            ) * 0.125
            s = jnp.where(mask, s, -1e9)
            m = jnp.max(s, axis=1, keepdims=True)
            e = jnp.exp(s - m)
            w = e / jnp.sum(e, axis=1, keepdims=True)
            ctx = lax.dot_general(
                w.astype(jnp.bfloat16), vfull,
                (((1,), (0,)), ((), ())), preferred_element_type=jnp.float32,
            )
            ctx_heads.append(ctx)
        ctx_b = jnp.concatenate(ctx_heads, axis=1).astype(jnp.bfloat16)
        out_ref[b] = lax.dot_general(
            ctx_b, wo_ref[...],
            (((1,), (0,)), ((), ())), preferred_element_type=jnp.float32,
        )


def kernel(x, Wq, K_ext, V_ext, Wo):
    bf16 = jnp.bfloat16
    xb = x.astype(bf16)
    wq = Wq.astype(bf16)
    wo = Wo.astype(bf16)
    k = jnp.transpose(K_ext.astype(bf16), (0, 2, 1, 3)).reshape(B * HQ, SKV, DH)
    v = jnp.transpose(V_ext.astype(bf16), (0, 2, 1, 3)).reshape(B * HQ, SKV, DH)
    kf, kl = k[:, :HALO], k[:, SKV - HALO:]
    vf, vl = v[:, :HALO], v[:, SKV - HALO:]

    halo = pltpu.VMEM((B * HQ, HALO, DH), bf16)
    return pl.pallas_call(
        _body,
        out_shape=jax.ShapeDtypeStruct((B, SQ, D), jnp.float32),
        in_specs=[pl.BlockSpec(memory_space=pltpu.VMEM)] * 9,
        out_specs=pl.BlockSpec(memory_space=pltpu.VMEM),
        scratch_shapes=[
            halo, halo, halo, halo,
            pltpu.SemaphoreType.DMA((4,)),
            pltpu.SemaphoreType.DMA((4,)),
        ],
        compiler_params=pltpu.CompilerParams(collective_id=0),
    )(xb, wq, k, v, kf, kl, vf, vl, wo)


# device time: 15887 ns/iter; 1.2465x vs baseline; 1.2465x over previous
import jax
import jax.numpy as jnp
from jax import lax
from jax.experimental import pallas as pl
from jax.experimental.pallas import tpu as pltpu

N_DEV = 4
B = 2
SQ = 256
SKV = 256
HQ = 4
DH = 64
D = 512
HD = HQ * DH
HALO = 128
WIN = 128
KW = SKV + 2 * HALO


def _body(x_ref, wq_ref, k_ref, v_ref, kf_ref, kl_ref, vf_ref, vl_ref,
          wo_ref, out_ref, kleft, kright, vleft, vright,
          send_sems, recv_sems):
    pos = lax.axis_index("i")
    left = (pos - 1) % N_DEV
    right = (pos + 1) % N_DEV

    barrier_sem = pltpu.get_barrier_semaphore()
    for nbr in (left, right):
        pl.semaphore_signal(
            barrier_sem, inc=1,
            device_id=(nbr,), device_id_type=pl.DeviceIdType.MESH,
        )
    pl.semaphore_wait(barrier_sem, 2)

    transfers = (
        (kl_ref, kleft, right),
        (vl_ref, vleft, right),
        (kf_ref, kright, left),
        (vf_ref, vright, left),
    )
    rdmas = []
    for idx, (src, dst, tgt) in enumerate(transfers):
        rdma = pltpu.make_async_remote_copy(
            src_ref=src, dst_ref=dst,
            send_sem=send_sems.at[idx], recv_sem=recv_sems.at[idx],
            device_id=(tgt,), device_id_type=pl.DeviceIdType.MESH,
        )
        rdma.start()
        rdmas.append(rdma)

    q = []
    for b in range(B):
        qb = lax.dot_general(
            x_ref[b], wq_ref[...],
            (((1,), (0,)), ((), ())), preferred_element_type=jnp.float32,
        )
        q.append((qb * 0.125).astype(jnp.bfloat16))

    i2 = lax.broadcasted_iota(jnp.int32, (SQ, KW), 0)
    j2 = lax.broadcasted_iota(jnp.int32, (SQ, KW), 1)
    ki = pos * SKV - HALO + j2
    mask = (jnp.abs(i2 + HALO - j2) <= WIN) & (ki >= 0) & (ki < N_DEV * SKV)
    bias = jnp.where(mask, 0.0, -1e9).astype(jnp.float32)

    for rdma in rdmas:
        rdma.wait()

    for b in range(B):
        kfull = jnp.concatenate([kleft[b], k_ref[b], kright[b]], axis=0)
        vfull = jnp.concatenate([vleft[b], v_ref[b], vright[b]], axis=0)
        ctx_heads = []
        for h in range(HQ):
            cols = slice(h * DH, (h + 1) * DH)
            s = lax.dot_general(
                q[b][:, cols], kfull[:, cols], (((1,), (1,)), ((), ())),
                preferred_element_type=jnp.float32,
            ) + bias
            e = jnp.exp(s)
            w = e * (1.0 / jnp.sum(e, axis=1, keepdims=True))
            ctx = lax.dot_general(
                w.astype(jnp.bfloat16), vfull[:, cols],
                (((1,), (0,)), ((), ())), preferred_element_type=jnp.float32,
            )
            ctx_heads.append(ctx)
        ctx_b = jnp.concatenate(ctx_heads, axis=1).astype(jnp.bfloat16)
        out_ref[b] = lax.dot_general(
            ctx_b, wo_ref[...],
            (((1,), (0,)), ((), ())), preferred_element_type=jnp.float32,
        )


def kernel(x, Wq, K_ext, V_ext, Wo):
    bf16 = jnp.bfloat16
    xb = x.astype(bf16)
    wq = Wq.astype(bf16)
    wo = Wo.astype(bf16)
    k = K_ext.astype(bf16).reshape(B, SKV, HD)
    v = V_ext.astype(bf16).reshape(B, SKV, HD)
    kf, kl = k[:, :HALO], k[:, SKV - HALO:]
    vf, vl = v[:, :HALO], v[:, SKV - HALO:]

    halo = pltpu.VMEM((B, HALO, HD), bf16)
    return pl.pallas_call(
        _body,
        out_shape=jax.ShapeDtypeStruct((B, SQ, D), jnp.float32),
        in_specs=[pl.BlockSpec(memory_space=pltpu.VMEM)] * 9,
        out_specs=pl.BlockSpec(memory_space=pltpu.VMEM),
        scratch_shapes=[
            halo, halo, halo, halo,
            pltpu.SemaphoreType.DMA((4,)),
            pltpu.SemaphoreType.DMA((4,)),
        ],
        compiler_params=pltpu.CompilerParams(collective_id=0),
    )(xb, wq, k, v, kf, kl, vf, vl, wo)
